# Initial kernel scaffold; baseline (speedup 1.0000x reference)
import jax
import jax.numpy as jnp
from jax.experimental import pallas as pl


def kernel(x, pe):
    raise NotImplementedError("write your pallas kernel here")


# speedup vs baseline: 1.4431x; 1.4431x over previous
import jax
import jax.numpy as jnp
from jax.experimental import pallas as pl

BATCH = 4
SEQ_LEN = 8192
D_MODEL = 768
BLK_S = 512


def _add_kernel(x_ref, pe_ref, o_ref):
    o_ref[0] = x_ref[0] + pe_ref[...]


def kernel(x, pe):
    n_s = SEQ_LEN // BLK_S
    return pl.pallas_call(
        _add_kernel,
        grid=(n_s, BATCH),
        in_specs=[
            pl.BlockSpec((1, BLK_S, D_MODEL), lambda s, b: (b, s, 0)),
            pl.BlockSpec((BLK_S, D_MODEL), lambda s, b: (s, 0)),
        ],
        out_specs=pl.BlockSpec((1, BLK_S, D_MODEL), lambda s, b: (b, s, 0)),
        out_shape=jax.ShapeDtypeStruct((BATCH, SEQ_LEN, D_MODEL), x.dtype),
    )(x, pe)


# speedup vs baseline: 1.6819x; 1.1655x over previous
import jax
import jax.numpy as jnp
from jax.experimental import pallas as pl

BATCH = 4
SEQ_LEN = 8192
D_MODEL = 768
BLK_S = 1024


def _add_kernel(x_ref, pe_ref, o_ref):
    o_ref[0] = x_ref[0] + pe_ref[...]


def kernel(x, pe):
    n_s = SEQ_LEN // BLK_S
    return pl.pallas_call(
        _add_kernel,
        grid=(n_s, BATCH),
        in_specs=[
            pl.BlockSpec((1, BLK_S, D_MODEL), lambda s, b: (b, s, 0)),
            pl.BlockSpec((BLK_S, D_MODEL), lambda s, b: (s, 0)),
        ],
        out_specs=pl.BlockSpec((1, BLK_S, D_MODEL), lambda s, b: (b, s, 0)),
        out_shape=jax.ShapeDtypeStruct((BATCH, SEQ_LEN, D_MODEL), x.dtype),
    )(x, pe)


# speedup vs baseline: 1.7928x; 1.0659x over previous
import jax
import jax.numpy as jnp
from jax.experimental import pallas as pl

BATCH = 4
SEQ_LEN = 8192
D_MODEL = 768
BLK_S = 2048


def _add_kernel(x_ref, pe_ref, o_ref):
    o_ref[0] = x_ref[0] + pe_ref[...]


def kernel(x, pe):
    n_s = SEQ_LEN // BLK_S
    return pl.pallas_call(
        _add_kernel,
        grid=(n_s, BATCH),
        in_specs=[
            pl.BlockSpec((1, BLK_S, D_MODEL), lambda s, b: (b, s, 0)),
            pl.BlockSpec((BLK_S, D_MODEL), lambda s, b: (s, 0)),
        ],
        out_specs=pl.BlockSpec((1, BLK_S, D_MODEL), lambda s, b: (b, s, 0)),
        out_shape=jax.ShapeDtypeStruct((BATCH, SEQ_LEN, D_MODEL), x.dtype),
    )(x, pe)


# speedup vs baseline: 1.7974x; 1.0026x over previous
import jax
import jax.numpy as jnp
from jax.experimental import pallas as pl
from jax.experimental.pallas import tpu as pltpu

BATCH = 4
SEQ_LEN = 8192
D_MODEL = 768
BLK_S = 2048


def _add_kernel(x_ref, pe_ref, o_ref):
    o_ref[0] = x_ref[0] + pe_ref[...]


def kernel(x, pe):
    n_s = SEQ_LEN // BLK_S
    return pl.pallas_call(
        _add_kernel,
        grid=(n_s, BATCH),
        in_specs=[
            pl.BlockSpec((1, BLK_S, D_MODEL), lambda s, b: (b, s, 0)),
            pl.BlockSpec((BLK_S, D_MODEL), lambda s, b: (s, 0)),
        ],
        out_specs=pl.BlockSpec((1, BLK_S, D_MODEL), lambda s, b: (b, s, 0)),
        out_shape=jax.ShapeDtypeStruct((BATCH, SEQ_LEN, D_MODEL), x.dtype),
        compiler_params=pltpu.CompilerParams(
            dimension_semantics=("parallel", "parallel")
        ),
    )(x, pe)
